# baseline (speedup 1.0000x reference)
import functools

import jax
import jax.numpy as jnp
from jax import lax
from jax.experimental import pallas as pl
from jax.experimental.pallas import tpu as pltpu
from jax.experimental.pallas import tpu_sc as plsc

_IB = 128


def _fc_body(emb_ref, w_ref, b_ref, out_ref):
    out_ref[...] = (
        jnp.dot(emb_ref[...], w_ref[...], preferred_element_type=jnp.float32)
        + b_ref[...]
    )


def _project_table(emb_table, W, b):
    V, _ = emb_table.shape
    C = W.shape[1]
    return pl.pallas_call(
        _fc_body,
        out_shape=jax.ShapeDtypeStruct((V, C), jnp.float32),
    )(emb_table, W, b.reshape(1, C))


@functools.lru_cache(maxsize=None)
def _make_gather(V, Cp, N):
    info = plsc.get_sparse_core_info()
    NC, NS = info.num_cores, info.num_subcores
    NW = NC * NS
    K = 128 // Cp
    nblk = N // _IB
    blk_per_w = nblk // NW
    assert nblk * _IB == N and blk_per_w * NW == nblk and blk_per_w % K == 0
    nsteps = blk_per_w // K
    rows_per_step = _IB * K * Cp // 128
    mesh = plsc.VectorSubcoreMesh(core_axis_name="c", subcore_axis_name="s")

    @functools.partial(
        pl.kernel,
        mesh=mesh,
        out_type=jax.ShapeDtypeStruct((N * Cp // 128, 128), jnp.float32),
        compiler_params=pltpu.CompilerParams(
            use_tc_tiling_on_sc=False, needs_layout_passes=False
        ),
        scratch_types=[
            pltpu.VMEM((K, _IB), jnp.int32),
            pltpu.VMEM((K, _IB, Cp), jnp.float32),
            pltpu.VMEM((128, 128), jnp.float32),
            pltpu.SemaphoreType.DMA,
            pltpu.SemaphoreType.DMA,
        ],
    )
    def gather_kernel(table_hbm, idx_hbm, out_hbm, idx_v, rows_v, pack_v,
                      isem, gsem):
        wid = lax.axis_index("s") * NC + lax.axis_index("c")
        base = wid * blk_per_w

        def step(i, carry):
            off = base + i * K
            pltpu.async_copy(idx_hbm.at[pl.ds(off, K)], idx_v, isem).wait()
            copies = []
            for j in range(K):
                copies.append(
                    pltpu.async_copy(
                        table_hbm.at[idx_v.at[j]], rows_v.at[j], gsem
                    )
                )
            for c in copies:
                c.wait()

            def repack(t8, carry2):
                for j in range(K):
                    for u in range(8):
                        pack_v[j * 16 + t8, pl.ds(u * 16, 16)] = rows_v[
                            j, t8 * 8 + u, :
                        ]
                return carry2

            lax.fori_loop(0, 16, repack, 0, unroll=False)
            pltpu.async_copy(
                pack_v,
                out_hbm.at[pl.ds(off * (_IB * Cp // 128), rows_per_step)],
                isem,
            ).wait()
            return carry

        lax.fori_loop(0, nsteps, step, 0, unroll=False)

    return gather_kernel


def _compact_body(x_ref, out_ref):
    br = lax.broadcasted_iota(jnp.int32, (128, 80), 0)
    bo = lax.broadcasted_iota(jnp.int32, (128, 80), 1)
    perm = ((br // 16 == bo // 10) & (br % 16 == bo % 10)).astype(jnp.float32)
    out_ref[...] = jnp.dot(
        x_ref[...], perm, preferred_element_type=jnp.float32
    )


@functools.lru_cache(maxsize=None)
def _make_compact(nrows, rblk):
    assert nrows % rblk == 0
    return pl.pallas_call(
        _compact_body,
        grid=(nrows // rblk,),
        in_specs=[pl.BlockSpec((rblk, 128), lambda i: (i, 0))],
        out_specs=pl.BlockSpec((rblk, 80), lambda i: (i, 0)),
        out_shape=jax.ShapeDtypeStruct((nrows, 80), jnp.float32),
    )


def kernel(x, emb_table, W, b):
    B, L = x.shape
    V, C = emb_table.shape[0], W.shape[1]
    N = B * L
    Cp = 16
    Wp = jnp.pad(W, ((0, 0), (0, Cp - C)))
    bp = jnp.pad(b, (0, Cp - C))
    table = _project_table(emb_table, Wp, bp)
    idx2d = x.reshape(N // _IB, _IB).astype(jnp.int32)
    padded = _make_gather(V, Cp, N)(table, idx2d)
    packed = _make_compact(N // 8, 4096)(padded)
    return packed.reshape(B, L, C)


# speedup vs baseline: 2.1246x; 2.1246x over previous
import functools

import jax
import jax.numpy as jnp
from jax import lax
from jax.experimental import pallas as pl
from jax.experimental.pallas import tpu as pltpu
from jax.experimental.pallas import tpu_sc as plsc

_IB = 128


def _fc_body(emb_ref, w_ref, b_ref, out_ref):
    out_ref[...] = (
        jnp.dot(emb_ref[...], w_ref[...], preferred_element_type=jnp.float32)
        + b_ref[...]
    )


def _project_table(emb_table, W, b):
    V, _ = emb_table.shape
    C = W.shape[1]
    return pl.pallas_call(
        _fc_body,
        out_shape=jax.ShapeDtypeStruct((V, C), jnp.float32),
    )(emb_table, W, b.reshape(1, C))


@functools.lru_cache(maxsize=None)
def _make_gather(V, Cp, N):
    info = plsc.get_sparse_core_info()
    NC, NS = info.num_cores, info.num_subcores
    NW = NC * NS
    K = 128 // Cp
    nblk = N // _IB
    blk_per_w = nblk // NW
    assert nblk * _IB == N and blk_per_w * NW == nblk and blk_per_w % K == 0
    nsteps = blk_per_w // K
    rows_per_step = _IB * K * Cp // 128
    mesh = plsc.VectorSubcoreMesh(core_axis_name="c", subcore_axis_name="s")

    @functools.partial(
        pl.kernel,
        mesh=mesh,
        out_type=jax.ShapeDtypeStruct((N * Cp // 128, 128), jnp.float32),
        compiler_params=pltpu.CompilerParams(
            use_tc_tiling_on_sc=False, needs_layout_passes=False
        ),
        scratch_types=[
            pltpu.VMEM((K, _IB), jnp.int32),
            pltpu.VMEM((K, _IB, Cp), jnp.float32),
            pltpu.VMEM((128, 128), jnp.float32),
            pltpu.SemaphoreType.DMA,
            pltpu.SemaphoreType.DMA,
        ],
    )
    def gather_kernel(table_hbm, idx_hbm, out_hbm, idx_v, rows_v, pack_v,
                      isem, gsem):
        wid = lax.axis_index("s") * NC + lax.axis_index("c")
        base = wid * blk_per_w

        def step(i, carry):
            off = base + i * K
            pltpu.async_copy(idx_hbm.at[pl.ds(off, K)], idx_v, isem).wait()
            copies = []
            for j in range(K):
                copies.append(
                    pltpu.async_copy(
                        table_hbm.at[idx_v.at[j]], rows_v.at[j], gsem
                    )
                )
            for c in copies:
                c.wait()

            def repack(t8, carry2):
                for j in range(K):
                    for u in range(8):
                        pack_v[j * 16 + t8, pl.ds(u * 16, 16)] = rows_v[
                            j, t8 * 8 + u, :
                        ]
                return carry2

            lax.fori_loop(0, 16, repack, 0, unroll=False)
            pltpu.async_copy(
                pack_v,
                out_hbm.at[pl.ds(off * (_IB * Cp // 128), rows_per_step)],
                isem,
            ).wait()
            return carry

        lax.fori_loop(0, nsteps, step, 0, unroll=False)

    return gather_kernel


def _tcompact_body(x_ref, out_ref):
    g2 = pl.program_id(1)
    d = g2 // 4
    colbase = 4096 * (g2 % 4)
    row = lax.broadcasted_iota(jnp.int32, (80, 128), 0)
    k = lax.broadcasted_iota(jnp.int32, (80, 128), 1)
    sel = (k == 16 * (row // 10) + row % 10).astype(jnp.float32)
    res = lax.dot_general(
        sel,
        x_ref[...],
        (((1,), (1,)), ((), ())),
        preferred_element_type=jnp.float32,
    )
    for u in range(8):
        out_ref[:, d, pl.ds(colbase + 512 * u, 512)] = res[
            10 * u : 10 * (u + 1), :
        ]


@functools.lru_cache(maxsize=None)
def _make_tcompact(L, B):
    N = L * B
    assert L % 8 == 0 and B % 4096 == 0
    return pl.pallas_call(
        _tcompact_body,
        grid=(L // 8, 32),
        in_specs=[pl.BlockSpec((512, 128), lambda i, j: (i * 32 + j, 0))],
        out_specs=pl.BlockSpec((10, 8, B), lambda i, j: (0, i, 0)),
        out_shape=jax.ShapeDtypeStruct((10, L, B), jnp.float32),
    )


def kernel(x, emb_table, W, b):
    B, L = x.shape
    V, C = emb_table.shape[0], W.shape[1]
    N = B * L
    Cp = 16
    Wp = jnp.pad(W, ((0, 0), (0, Cp - C)))
    bp = jnp.pad(b, (0, Cp - C))
    table = _project_table(emb_table, Wp, bp)
    xq = x.T.reshape(N // 4096, 8, 512).transpose(0, 2, 1)
    idx2d = xq.reshape(N // _IB, _IB).astype(jnp.int32)
    padded = _make_gather(V, Cp, N)(table, idx2d)
    packedT = _make_tcompact(L, B)(padded)
    return packedT.transpose(2, 1, 0)


# speedup vs baseline: 2.4134x; 1.1359x over previous
import functools

import jax
import jax.numpy as jnp
from jax import lax
from jax.experimental import pallas as pl
from jax.experimental.pallas import tpu as pltpu
from jax.experimental.pallas import tpu_sc as plsc

_IB = 128


def _fc_body(emb_ref, w_ref, b_ref, out_ref):
    out_ref[...] = (
        jnp.dot(emb_ref[...], w_ref[...], preferred_element_type=jnp.float32)
        + b_ref[...]
    )


def _project_table(emb_table, W, b):
    V, _ = emb_table.shape
    C = W.shape[1]
    return pl.pallas_call(
        _fc_body,
        out_shape=jax.ShapeDtypeStruct((V, C), jnp.float32),
    )(emb_table, W, b.reshape(1, C))


@functools.lru_cache(maxsize=None)
def _make_gather(V, Cp, N):
    info = plsc.get_sparse_core_info()
    NC, NS = info.num_cores, info.num_subcores
    NW = NC * NS
    K = 128 // Cp
    nblk = N // _IB
    blk_per_w = nblk // NW
    assert nblk * _IB == N and blk_per_w * NW == nblk and blk_per_w % K == 0
    nsteps = blk_per_w // K
    rows_per_step = _IB * K * Cp // 128
    mesh = plsc.VectorSubcoreMesh(core_axis_name="c", subcore_axis_name="s")

    assert nsteps % 2 == 0 and nsteps >= 4

    @functools.partial(
        pl.kernel,
        mesh=mesh,
        out_type=jax.ShapeDtypeStruct((N * Cp // 128, 128), jnp.float32),
        compiler_params=pltpu.CompilerParams(
            use_tc_tiling_on_sc=False, needs_layout_passes=False
        ),
        scratch_types=[
            pltpu.VMEM((2, K, _IB), jnp.int32),
            pltpu.VMEM((2, K, _IB, Cp), jnp.float32),
            pltpu.VMEM((2, 128, 128), jnp.float32),
            pltpu.SemaphoreType.DMA,
            pltpu.SemaphoreType.DMA,
            pltpu.SemaphoreType.DMA,
            pltpu.SemaphoreType.DMA,
            pltpu.SemaphoreType.DMA,
            pltpu.SemaphoreType.DMA,
        ],
    )
    def gather_kernel(table_hbm, idx_hbm, out_hbm, idx_v, rows_v, pack_v,
                      isem0, isem1, gsem0, gsem1, osem0, osem1):
        isems = (isem0, isem1)
        gsems = (gsem0, gsem1)
        osems = (osem0, osem1)
        wid = lax.axis_index("s") * NC + lax.axis_index("c")
        base = wid * blk_per_w

        def idx_copy(p, off):
            return pltpu.make_async_copy(
                idx_hbm.at[pl.ds(off, K)], idx_v.at[p], isems[p]
            )

        def gather_copy(p, j):
            return pltpu.make_async_copy(
                table_hbm.at[idx_v.at[p, j]], rows_v.at[p, j], gsems[p]
            )

        def out_copy(p, off):
            return pltpu.make_async_copy(
                pack_v.at[p],
                out_hbm.at[pl.ds(off * (_IB * Cp // 128), rows_per_step)],
                osems[p],
            )

        def repack(p):
            def body(t8, carry2):
                for j in range(K):
                    for u in range(8):
                        pack_v[p, j * 16 + t8, pl.ds(u * 16, 16)] = rows_v[
                            p, j, t8 * 8 + u, :
                        ]
                return carry2

            lax.fori_loop(0, 16, body, 0, unroll=False)

        idx_copy(0, base).start()
        idx_copy(0, base).wait()
        for j in range(K):
            gather_copy(0, j).start()
        idx_copy(1, base + K).start()

        def phase(p, i2):
            s = 2 * i2 + p
            off = base + s * K
            for j in range(K):
                gather_copy(p, j).wait()

            @pl.when(s + 1 < nsteps)
            def _():
                idx_copy(1 - p, base + (s + 1) * K).wait()
                for j in range(K):
                    gather_copy(1 - p, j).start()

            @pl.when(s + 2 < nsteps)
            def _():
                idx_copy(p, base + (s + 2) * K).start()

            @pl.when(s >= 2)
            def _():
                out_copy(p, base + (s - 2) * K).wait()

            repack(p)
            out_copy(p, off).start()

        def pair(i2, carry):
            phase(0, i2)
            phase(1, i2)
            return carry

        lax.fori_loop(0, nsteps // 2, pair, 0, unroll=False)
        out_copy(0, base + (nsteps - 2) * K).wait()
        out_copy(1, base + (nsteps - 1) * K).wait()

    return gather_kernel


def _tcompact_body(x_ref, out_ref):
    g2 = pl.program_id(1)
    d = g2 // 4
    colbase = 4096 * (g2 % 4)
    row = lax.broadcasted_iota(jnp.int32, (80, 128), 0)
    k = lax.broadcasted_iota(jnp.int32, (80, 128), 1)
    sel = (k == 16 * (row // 10) + row % 10).astype(jnp.float32)
    res = lax.dot_general(
        sel,
        x_ref[...],
        (((1,), (1,)), ((), ())),
        preferred_element_type=jnp.float32,
    )
    for u in range(8):
        out_ref[:, d, pl.ds(colbase + 512 * u, 512)] = res[
            10 * u : 10 * (u + 1), :
        ]


@functools.lru_cache(maxsize=None)
def _make_tcompact(L, B):
    N = L * B
    assert L % 8 == 0 and B % 4096 == 0
    return pl.pallas_call(
        _tcompact_body,
        grid=(L // 8, 32),
        in_specs=[pl.BlockSpec((512, 128), lambda i, j: (i * 32 + j, 0))],
        out_specs=pl.BlockSpec((10, 8, B), lambda i, j: (0, i, 0)),
        out_shape=jax.ShapeDtypeStruct((10, L, B), jnp.float32),
    )


def kernel(x, emb_table, W, b):
    B, L = x.shape
    V, C = emb_table.shape[0], W.shape[1]
    N = B * L
    Cp = 16
    Wp = jnp.pad(W, ((0, 0), (0, Cp - C)))
    bp = jnp.pad(b, (0, Cp - C))
    table = _project_table(emb_table, Wp, bp)
    xq = x.T.reshape(N // 4096, 8, 512).transpose(0, 2, 1)
    idx2d = xq.reshape(N // _IB, _IB).astype(jnp.int32)
    padded = _make_gather(V, Cp, N)(table, idx2d)
    packedT = _make_tcompact(L, B)(padded)
    return packedT.transpose(2, 1, 0)
